# baseline (speedup 1.0000x reference)
import functools

import jax
import jax.numpy as jnp
from jax import lax
from jax.experimental import pallas as pl
from jax.experimental.pallas import tpu as pltpu
from jax.experimental.pallas import tpu_sc as plsc

NC = 2
NS = 16
NW = NC * NS
LANES = 16
BLK = 128


def _matmul_body(x_ref, w_ref, o_ref):
    o_ref[0] = jnp.dot(x_ref[...], w_ref[0], preferred_element_type=jnp.float32)


def _finalize_body(agg_ref, deg_ref, xroot_ref, bias_ref, o_ref):
    a = agg_ref[0] + agg_ref[1]
    ones_b = jnp.ones((NW, a.shape[1]), jnp.float32)
    d = lax.dot_general(deg_ref[...], ones_b, (((0,), (0,)), ((), ())),
                        preferred_element_type=jnp.float32)
    d = jnp.maximum(d, 1.0)
    o_ref[...] = jnp.maximum(a / d + xroot_ref[...] + bias_ref[...], 0.0)


def _chunks(total, step):
    out = []
    r0 = 0
    while r0 < total:
        out.append((r0, min(step, total - r0)))
        r0 += min(step, total - r0)
    return out


def _make_sc_agg(N, H, NPAD, GPT):
    RPT = NPAD // NS

    mesh = plsc.VectorSubcoreMesh(
        core_axis_name="c", subcore_axis_name="s", num_cores=NC, num_subcores=NS)

    @functools.partial(
        pl.kernel,
        out_type=[
            jax.ShapeDtypeStruct((NC, NPAD, H), jnp.float32),
            jax.ShapeDtypeStruct((NW, NPAD // BLK, BLK), jnp.float32),
        ],
        mesh=mesh,
        scratch_types=[
            pltpu.VMEM((2, BLK), jnp.int32),
            pltpu.VMEM((2, BLK), jnp.int32),
            pltpu.VMEM((BLK, H), jnp.float32),
            pltpu.VMEM((BLK, H), jnp.float32),
            pltpu.VMEM((NPAD // BLK, BLK), jnp.float32),
            pltpu.VMEM_SHARED((NPAD, H), jnp.float32),
            pltpu.SemaphoreType.DMA,
            pltpu.SemaphoreType.DMA,
            pltpu.SemaphoreType.DMA,
            pltpu.SemaphoreType.DMA,
        ],
        compiler_params=pltpu.CompilerParams(needs_layout_passes=False),
        name="sc_agg",
    )
    def sc_agg(xall, gidx, didx, z1, agg_out, deg_out,
               gidx_v, didx_v, rows0, rows1, deg_v, agg_s,
               semi0, semi1, semr0, semr1):
        c = lax.axis_index("c")
        s = lax.axis_index("s")
        wid = c * NS + s
        row0 = s * RPT

        zeros16 = jnp.zeros((LANES,), jnp.float32)
        ones16 = jnp.ones((LANES,), jnp.float32)

        ia = pltpu.async_copy(gidx.at[wid, 0], gidx_v.at[0], semi0)
        ib = pltpu.async_copy(didx.at[wid, 0], didx_v.at[0], semi0)
        ic = pltpu.async_copy(gidx.at[wid, 1], gidx_v.at[1], semi1)
        id_ = pltpu.async_copy(didx.at[wid, 1], didx_v.at[1], semi1)

        def zero_step(i, carry):
            for k in range(BLK // LANES):
                deg_v[i, pl.ds(k * LANES, LANES)] = zeros16
            return carry

        lax.fori_loop(0, NPAD // BLK, zero_step, 0)

        pltpu.sync_copy(z1, rows0)
        for r, sz in _chunks(RPT, BLK):
            pltpu.sync_copy(rows0.at[pl.ds(0, sz)], agg_s.at[pl.ds(row0 + r, sz)])
        plsc.subcore_barrier()

        ia.wait()
        ib.wait()
        pltpu.async_copy(xall.at[gidx_v.at[0]], rows0, semr0)
        ic.wait()
        id_.wait()

        def drain_idx(sem):
            pltpu.make_async_copy(gidx.at[0, 0], gidx_v.at[0], sem).wait()
            pltpu.make_async_copy(didx.at[0, 0], didx_v.at[0], sem).wait()

        def drain_rows(buf, sem):
            pltpu.make_async_copy(xall.at[pl.ds(0, BLK)], buf, sem).wait()

        def scatter_block(buf, slot):
            pltpu.sync_copy(buf, agg_s.at[didx_v.at[slot]], add=True)
            for k in range(BLK // LANES):
                dvec = didx_v[slot, pl.ds(k * LANES, LANES)]
                plsc.addupdate_scatter(
                    deg_v, [lax.shift_right_logical(dvec, 7),
                            lax.bitwise_and(dvec, 127)], ones16)

        def pair_step(p, carry):
            g0 = 2 * p
            pltpu.async_copy(xall.at[gidx_v.at[1]], rows1, semr1)
            drain_rows(rows0, semr0)
            scatter_block(rows0, 0)
            pltpu.async_copy(gidx.at[wid, g0 + 2], gidx_v.at[0], semi0)
            pltpu.async_copy(didx.at[wid, g0 + 2], didx_v.at[0], semi0)
            drain_idx(semi0)
            pltpu.async_copy(xall.at[gidx_v.at[0]], rows0, semr0)
            drain_rows(rows1, semr1)
            scatter_block(rows1, 1)
            pltpu.async_copy(gidx.at[wid, g0 + 3], gidx_v.at[1], semi1)
            pltpu.async_copy(didx.at[wid, g0 + 3], didx_v.at[1], semi1)
            drain_idx(semi1)
            return carry

        lax.fori_loop(0, GPT // 2, pair_step, 0)
        drain_rows(rows0, semr0)
        plsc.subcore_barrier()

        for r, sz in _chunks(RPT, BLK):
            pltpu.sync_copy(agg_s.at[pl.ds(row0 + r, sz)], rows0.at[pl.ds(0, sz)])
            pltpu.sync_copy(rows0.at[pl.ds(0, sz)], agg_out.at[c, pl.ds(row0 + r, sz)])
        pltpu.sync_copy(deg_v, deg_out.at[wid])

    return sc_agg


def _make_sc_score(N, H, R, TPT, TB):
    mesh = plsc.VectorSubcoreMesh(
        core_axis_name="c", subcore_axis_name="s", num_cores=NC, num_subcores=NS)

    @functools.partial(
        pl.kernel,
        out_type=jax.ShapeDtypeStruct((NW * TPT,), jnp.float32),
        mesh=mesh,
        scratch_types=[
            pltpu.VMEM((TB + 1, BLK), jnp.int32),
            pltpu.VMEM((TB + 1, BLK), jnp.int32),
            pltpu.VMEM((TB + 1, BLK), jnp.int32),
            pltpu.VMEM((BLK, H), jnp.float32),
            pltpu.VMEM((BLK, H), jnp.float32),
            pltpu.VMEM((BLK, H), jnp.float32),
            pltpu.VMEM((BLK, H), jnp.float32),
            pltpu.VMEM((BLK, H), jnp.float32),
            pltpu.VMEM((BLK, H), jnp.float32),
            pltpu.VMEM((TPT,), jnp.float32),
            pltpu.VMEM((LANES * LANES,), jnp.float32),
            pltpu.SemaphoreType.DMA,
            pltpu.SemaphoreType.DMA,
        ],
        compiler_params=pltpu.CompilerParams(needs_layout_passes=False),
        name="sc_score",
    )
    def sc_score(h, rel, hidx, tidx, ridx, out,
                 hidx_v, tidx_v, ridx_v, hrow0, hrow1, trow0, trow1,
                 rrow0, rrow1, sc_v, tmp_v, semg0, semg1):
        c = lax.axis_index("c")
        s = lax.axis_index("s")
        wid = c * NS + s

        pltpu.sync_copy(hidx.at[wid], hidx_v)
        pltpu.sync_copy(tidx.at[wid], tidx_v)
        pltpu.sync_copy(ridx.at[wid], ridx_v)

        def issue(g, hb, tb, rb, sem):
            pltpu.async_copy(h.at[hidx_v.at[g]], hb, sem)
            pltpu.async_copy(h.at[tidx_v.at[g]], tb, sem)
            pltpu.async_copy(rel.at[ridx_v.at[g]], rb, sem)

        def drain(hb, tb, rb, sem):
            pltpu.make_async_copy(h.at[pl.ds(0, BLK)], hb, sem).wait()
            pltpu.make_async_copy(h.at[pl.ds(0, BLK)], tb, sem).wait()
            pltpu.make_async_copy(h.at[pl.ds(0, BLK)], rb, sem).wait()

        iota16 = jnp.arange(LANES, dtype=jnp.int32) * LANES

        def compute(g, hb, tb, rb):
            def sub(b, carry2):
                for j in range(LANES):
                    row = b * LANES + j
                    acc = hb[row, 0:LANES] * tb[row, 0:LANES] * rb[row, 0:LANES]
                    for v in range(1, H // LANES):
                        sl = pl.ds(v * LANES, LANES)
                        acc = acc + hb[row, sl] * tb[row, sl] * rb[row, sl]
                    tmp_v[pl.ds(j * LANES, LANES)] = acc
                svec = plsc.load_gather(tmp_v, [iota16])
                for k in range(1, LANES):
                    svec = svec + plsc.load_gather(tmp_v, [iota16 + k])
                sc_v[pl.ds(g * BLK + b * LANES, LANES)] = svec
                return carry2

            lax.fori_loop(0, BLK // LANES, sub, 0)

        issue(0, hrow0, trow0, rrow0, semg0)

        def pair_step(p, carry):
            g0 = 2 * p
            issue(g0 + 1, hrow1, trow1, rrow1, semg1)
            drain(hrow0, trow0, rrow0, semg0)
            compute(g0, hrow0, trow0, rrow0)
            issue(g0 + 2, hrow0, trow0, rrow0, semg0)
            drain(hrow1, trow1, rrow1, semg1)
            compute(g0 + 1, hrow1, trow1, rrow1)
            return carry

        lax.fori_loop(0, TB // 2, pair_step, 0)
        drain(hrow0, trow0, rrow0, semg0)
        pltpu.sync_copy(sc_v, out.at[pl.ds(wid * TPT, TPT)])

    return sc_score


def _pad_reshape(a, total, fill, shape):
    pad = total - a.shape[0]
    a = jnp.concatenate([a, jnp.full((pad,), fill, a.dtype)])
    return a.reshape(shape)


def _slab_pad(a, nblk, extra, fill):
    a = _pad_reshape(a, NW * nblk * BLK, fill, (NW, nblk, BLK))
    tail = jnp.full((NW, extra, BLK), fill, a.dtype)
    return jnp.concatenate([a, tail], axis=1)


def kernel(node_ids, edge_index, edge_type, head_idx, rel_idx, tail_idx,
           emb_table, W, W_root, bias, rel_table):
    N, D = emb_table.shape
    R, _, H = W.shape
    E = edge_type.shape[0]
    T = head_idx.shape[0]

    NPAD = -(-N // (NS * BLK)) * NS * BLK

    x = jnp.take(emb_table, node_ids, axis=0)
    x = jnp.concatenate([x, jnp.zeros((NPAD - N, D), jnp.float32)], axis=0)
    Wcat = jnp.concatenate([W, W_root[None]], axis=0)

    BN = 1024
    xall = pl.pallas_call(
        _matmul_body,
        grid=(R + 1, NPAD // BN),
        in_specs=[
            pl.BlockSpec((BN, D), lambda r, i: (i, 0)),
            pl.BlockSpec((1, D, H), lambda r, i: (r, 0, 0)),
        ],
        out_specs=pl.BlockSpec((1, BN, H), lambda r, i: (r, i, 0)),
        out_shape=jax.ShapeDtypeStruct((R + 1, NPAD, H), jnp.float32),
    )(x, Wcat)
    xall_flat = xall.reshape((R + 1) * NPAD, H)

    GPT = -(-E // (NW * BLK))
    GPT = GPT + (GPT % 2)

    src = edge_index[0]
    dst = edge_index[1]
    gidx = _slab_pad(edge_type * NPAD + src, GPT, 2, R * NPAD)
    didx = _slab_pad(dst, GPT, 2, N)

    z1 = jnp.zeros((BLK, H), jnp.float32)

    agg, deg = _make_sc_agg(N, H, NPAD, GPT)(xall_flat, gidx, didx, z1)
    deg = deg.reshape(NW, NPAD)

    BN2 = 1280
    h = pl.pallas_call(
        _finalize_body,
        grid=(NPAD // BN2,),
        in_specs=[
            pl.BlockSpec((NC, BN2, H), lambda i: (0, i, 0)),
            pl.BlockSpec((NW, BN2), lambda i: (0, i)),
            pl.BlockSpec((BN2, H), lambda i: (i, 0)),
            pl.BlockSpec((1, H), lambda i: (0, 0)),
        ],
        out_specs=pl.BlockSpec((BN2, H), lambda i: (i, 0)),
        out_shape=jax.ShapeDtypeStruct((NPAD, H), jnp.float32),
    )(agg, deg, xall[R], bias.reshape(1, H))

    TB = -(-T // (NW * BLK))
    TB = TB + (TB % 2)
    TPT = TB * BLK
    hidx = _slab_pad(head_idx, TB, 1, 0)
    tidx = _slab_pad(tail_idx, TB, 1, 0)
    ridx = _slab_pad(rel_idx, TB, 1, 0)

    scores = _make_sc_score(N, H, R, TPT, TB)(h, rel_table, hidx, tidx, ridx)
    return scores[:T]


# speedup vs baseline: 2.3801x; 2.3801x over previous
import functools

import jax
import jax.numpy as jnp
from jax import lax
from jax.experimental import pallas as pl
from jax.experimental.pallas import tpu as pltpu
from jax.experimental.pallas import tpu_sc as plsc

NC = 2
NS = 16
NW = NC * NS
LANES = 16
BLK = 128


def _matmul_body(x_ref, w_ref, o_ref):
    o_ref[0] = jnp.dot(x_ref[...], w_ref[0], preferred_element_type=jnp.float32)


def _finalize_body(agg_ref, deg_ref, xroot_ref, bias_ref, o_ref):
    a = agg_ref[0] + agg_ref[1]
    ones_b = jnp.ones((NW, a.shape[1]), jnp.float32)
    d = lax.dot_general(deg_ref[...], ones_b, (((0,), (0,)), ((), ())),
                        preferred_element_type=jnp.float32)
    d = jnp.maximum(d, 1.0)
    o_ref[...] = jnp.maximum(a / d + xroot_ref[...] + bias_ref[...], 0.0)


def _chunks(total, step):
    out = []
    r0 = 0
    while r0 < total:
        out.append((r0, min(step, total - r0)))
        r0 += min(step, total - r0)
    return out


def _make_sc_agg(N, H, NPAD, GPT):
    RPT = NPAD // NS

    mesh = plsc.VectorSubcoreMesh(
        core_axis_name="c", subcore_axis_name="s", num_cores=NC, num_subcores=NS)

    @functools.partial(
        pl.kernel,
        out_type=[
            jax.ShapeDtypeStruct((NC, NPAD, H), jnp.float32),
            jax.ShapeDtypeStruct((NW, NPAD // BLK, BLK), jnp.float32),
        ],
        mesh=mesh,
        scratch_types=[
            pltpu.VMEM((GPT, BLK), jnp.int32),
            pltpu.VMEM((GPT, BLK), jnp.int32),
            pltpu.VMEM((BLK, H), jnp.float32),
            pltpu.VMEM((NPAD // BLK, BLK), jnp.float32),
            pltpu.VMEM_SHARED((NPAD, H), jnp.float32),
            pltpu.SemaphoreType.DMA,
        ],
        compiler_params=pltpu.CompilerParams(needs_layout_passes=False),
        name="sc_agg",
    )
    def sc_agg(xall, gidx, didx, z1, agg_out, deg_out,
               gidx_v, didx_v, rows_v, deg_v, agg_s, sem):
        c = lax.axis_index("c")
        s = lax.axis_index("s")
        wid = c * NS + s
        row0 = s * RPT

        zeros16 = jnp.zeros((LANES,), jnp.float32)
        ones16 = jnp.ones((LANES,), jnp.float32)

        def zero_step(i, carry):
            for k in range(BLK // LANES):
                deg_v[i, pl.ds(k * LANES, LANES)] = zeros16
            return carry

        lax.fori_loop(0, NPAD // BLK, zero_step, 0)

        pltpu.sync_copy(z1, rows_v)
        for r, sz in _chunks(RPT, BLK):
            pltpu.sync_copy(rows_v.at[pl.ds(0, sz)], agg_s.at[pl.ds(row0 + r, sz)])

        pltpu.sync_copy(gidx.at[wid], gidx_v)
        pltpu.sync_copy(didx.at[wid], didx_v)
        plsc.subcore_barrier()

        def blk_step(g, carry):
            pltpu.async_copy(xall.at[gidx_v.at[g]], rows_v, sem).wait()
            pltpu.sync_copy(rows_v, agg_s.at[didx_v.at[g]], add=True)
            for k in range(BLK // LANES):
                dvec = didx_v[g, pl.ds(k * LANES, LANES)]
                plsc.addupdate_scatter(
                    deg_v, [lax.shift_right_logical(dvec, 7),
                            lax.bitwise_and(dvec, 127)], ones16)
            return carry

        lax.fori_loop(0, GPT, blk_step, 0)
        plsc.subcore_barrier()

        for r, sz in _chunks(RPT, BLK):
            pltpu.sync_copy(agg_s.at[pl.ds(row0 + r, sz)], rows_v.at[pl.ds(0, sz)])
            pltpu.sync_copy(rows_v.at[pl.ds(0, sz)], agg_out.at[c, pl.ds(row0 + r, sz)])
        pltpu.sync_copy(deg_v, deg_out.at[wid])

    return sc_agg


def _make_sc_score(N, H, R, TPT, TB):
    mesh = plsc.VectorSubcoreMesh(
        core_axis_name="c", subcore_axis_name="s", num_cores=NC, num_subcores=NS)

    @functools.partial(
        pl.kernel,
        out_type=jax.ShapeDtypeStruct((NW * TPT,), jnp.float32),
        mesh=mesh,
        scratch_types=[
            pltpu.VMEM((TB, BLK), jnp.int32),
            pltpu.VMEM((TB, BLK), jnp.int32),
            pltpu.VMEM((TB, BLK), jnp.int32),
            pltpu.VMEM((BLK, H), jnp.float32),
            pltpu.VMEM((BLK, H), jnp.float32),
            pltpu.VMEM((R * H,), jnp.float32),
            pltpu.VMEM((TPT,), jnp.float32),
            pltpu.VMEM((LANES * LANES,), jnp.float32),
            pltpu.SemaphoreType.DMA,
        ],
        compiler_params=pltpu.CompilerParams(needs_layout_passes=False),
        name="sc_score",
    )
    def sc_score(h, rel, hidx, tidx, ridx, out,
                 hidx_v, tidx_v, ridx_v, hrow, trow, rel_v, sc_v, tmp_v, sem):
        c = lax.axis_index("c")
        s = lax.axis_index("s")
        wid = c * NS + s

        pltpu.sync_copy(hidx.at[wid], hidx_v)
        pltpu.sync_copy(tidx.at[wid], tidx_v)
        pltpu.sync_copy(ridx.at[wid], ridx_v)
        pltpu.sync_copy(rel, rel_v)

        iota16 = jnp.arange(LANES, dtype=jnp.int32) * LANES
        iota1 = jnp.arange(LANES, dtype=jnp.int32)

        def blk_step(g, carry):
            d1 = pltpu.async_copy(h.at[hidx_v.at[g]], hrow, sem)
            d2 = pltpu.async_copy(h.at[tidx_v.at[g]], trow, sem)
            d1.wait()
            d2.wait()

            def sub(b, carry2):
                rvec = ridx_v[g, pl.ds(b * LANES, LANES)] * H
                for j in range(LANES):
                    row = b * LANES + j
                    rj = jnp.broadcast_to(rvec[j], (LANES,)) + iota1
                    acc = (hrow[row, 0:LANES] * trow[row, 0:LANES]
                           * plsc.load_gather(rel_v, [rj]))
                    for v in range(1, H // LANES):
                        sl = pl.ds(v * LANES, LANES)
                        acc = acc + (hrow[row, sl] * trow[row, sl]
                                     * plsc.load_gather(rel_v, [rj + v * LANES]))
                    tmp_v[pl.ds(j * LANES, LANES)] = acc
                svec = plsc.load_gather(tmp_v, [iota16])
                for k in range(1, LANES):
                    svec = svec + plsc.load_gather(tmp_v, [iota16 + k])
                sc_v[pl.ds(g * BLK + b * LANES, LANES)] = svec
                return carry2

            lax.fori_loop(0, BLK // LANES, sub, 0)
            return carry

        lax.fori_loop(0, TB, blk_step, 0)
        pltpu.sync_copy(sc_v, out.at[pl.ds(wid * TPT, TPT)])

    return sc_score


def _pad_reshape(a, total, fill, shape):
    pad = total - a.shape[0]
    a = jnp.concatenate([a, jnp.full((pad,), fill, a.dtype)])
    return a.reshape(shape)


def kernel(node_ids, edge_index, edge_type, head_idx, rel_idx, tail_idx,
           emb_table, W, W_root, bias, rel_table):
    N, D = emb_table.shape
    R, _, H = W.shape
    E = edge_type.shape[0]
    T = head_idx.shape[0]

    NPAD = -(-N // (NS * BLK)) * NS * BLK

    x = jnp.take(emb_table, node_ids, axis=0)
    x = jnp.concatenate([x, jnp.zeros((NPAD - N, D), jnp.float32)], axis=0)
    Wcat = jnp.concatenate([W, W_root[None]], axis=0)

    BN = 1024
    xall = pl.pallas_call(
        _matmul_body,
        grid=(R + 1, NPAD // BN),
        in_specs=[
            pl.BlockSpec((BN, D), lambda r, i: (i, 0)),
            pl.BlockSpec((1, D, H), lambda r, i: (r, 0, 0)),
        ],
        out_specs=pl.BlockSpec((1, BN, H), lambda r, i: (r, i, 0)),
        out_shape=jax.ShapeDtypeStruct((R + 1, NPAD, H), jnp.float32),
    )(x, Wcat)
    xall_flat = xall.reshape((R + 1) * NPAD, H)

    GPT = -(-E // (NW * BLK))
    EPAD = NW * GPT * BLK

    src = edge_index[0]
    dst = edge_index[1]
    gidx = _pad_reshape(edge_type * NPAD + src, EPAD, R * NPAD, (NW, GPT, BLK))
    didx = _pad_reshape(dst, EPAD, N, (NW, GPT, BLK))

    z1 = jnp.zeros((BLK, H), jnp.float32)

    agg, deg = _make_sc_agg(N, H, NPAD, GPT)(xall_flat, gidx, didx, z1)
    deg = deg.reshape(NW, NPAD)

    BN2 = 1280
    h = pl.pallas_call(
        _finalize_body,
        grid=(NPAD // BN2,),
        in_specs=[
            pl.BlockSpec((NC, BN2, H), lambda i: (0, i, 0)),
            pl.BlockSpec((NW, BN2), lambda i: (0, i)),
            pl.BlockSpec((BN2, H), lambda i: (i, 0)),
            pl.BlockSpec((1, H), lambda i: (0, 0)),
        ],
        out_specs=pl.BlockSpec((BN2, H), lambda i: (i, 0)),
        out_shape=jax.ShapeDtypeStruct((NPAD, H), jnp.float32),
    )(agg, deg, xall[R], bias.reshape(1, H))

    TB = -(-T // (NW * BLK))
    TPT = TB * BLK
    TPAD = NW * TPT
    hidx = _pad_reshape(head_idx, TPAD, 0, (NW, TB, BLK))
    tidx = _pad_reshape(tail_idx, TPAD, 0, (NW, TB, BLK))
    ridx = _pad_reshape(rel_idx, TPAD, 0, (NW, TB, BLK))

    scores = _make_sc_score(N, H, R, TPT, TB)(
        h, rel_table.reshape(R * H), hidx, tidx, ridx)
    return scores[:T]
